# initial kernel scaffold (unmeasured)
import jax
import jax.numpy as jnp
from jax import lax
from jax.experimental import pallas as pl
from jax.experimental.pallas import tpu as pltpu

N_DEV = 8


def kernel(x, w_mat):
    m_total, k_shard = x.shape
    k_total, n = w_mat.shape
    m_per = m_total // N_DEV

    def body(x_ref, w_hbm, out_ref, xg_ref, w_buf, send_sems, recv_sems, w_sems):
        my = lax.axis_index("i")

        barrier = pltpu.get_barrier_semaphore()
        for d in range(1, N_DEV):
            pl.semaphore_signal(
                barrier, inc=1,
                device_id=((my + d) % N_DEV,),
                device_id_type=pl.DeviceIdType.MESH,
            )
        pl.semaphore_wait(barrier, N_DEV - 1)

        xg_ref[0] = x_ref[pl.ds(my * m_per, m_per), :]

        rdmas = []
        for d in range(1, N_DEV):
            tgt = (my + d) % N_DEV
            rdma = pltpu.make_async_remote_copy(
                src_ref=x_ref.at[pl.ds(tgt * m_per, m_per), :],
                dst_ref=xg_ref.at[N_DEV - d],
                send_sem=send_sems.at[d - 1],
                recv_sem=recv_sems.at[d - 1],
                device_id=(tgt,),
                device_id_type=pl.DeviceIdType.MESH,
            )
            rdma.start()
            rdmas.append(rdma)

        def fetch_w(sidx):
            kb = (my + sidx) % N_DEV
            cp = pltpu.make_async_copy(
                w_hbm.at[pl.ds(kb * k_shard, k_shard), :],
                w_buf.at[sidx % 2],
                w_sems.at[sidx % 2],
            )
            cp.start()
            return cp

        w_cp = fetch_w(0)
        for sidx in range(N_DEV):
            if sidx > 0:
                rdmas[(N_DEV - sidx) - 1].wait_recv()
            w_cp.wait()
            contrib = jnp.dot(
                xg_ref[sidx], w_buf[sidx % 2],
                preferred_element_type=jnp.float32,
            )
            if sidx + 1 < N_DEV:
                w_cp = fetch_w(sidx + 1)
            if sidx == 0:
                out_ref[...] = contrib
            else:
                out_ref[...] += contrib

        out_ref[...] = jnp.maximum(out_ref[...], 0.0)

        for rdma in rdmas:
            rdma.wait_send()

    return pl.pallas_call(
        body,
        out_shape=jax.ShapeDtypeStruct((m_per, n), jnp.float32),
        in_specs=[
            pl.BlockSpec(memory_space=pltpu.VMEM),
            pl.BlockSpec(memory_space=pltpu.ANY),
        ],
        out_specs=pl.BlockSpec(memory_space=pltpu.VMEM),
        scratch_shapes=[
            pltpu.VMEM((N_DEV, m_per, k_shard), jnp.bfloat16),
            pltpu.VMEM((2, k_shard, n), jnp.bfloat16),
            pltpu.SemaphoreType.DMA((N_DEV - 1,)),
            pltpu.SemaphoreType.DMA((N_DEV - 1,)),
            pltpu.SemaphoreType.DMA((2,)),
        ],
        compiler_params=pltpu.CompilerParams(collective_id=0),
    )(x, w_mat)


# baseline (device time: 140015 ns/iter reference)
import jax
import jax.numpy as jnp
from jax import lax
from jax.experimental import pallas as pl
from jax.experimental.pallas import tpu as pltpu

N_DEV = 8
N_CHUNK = 2


def kernel(x, w_mat):
    m_total, k_shard = x.shape
    k_total, n = w_mat.shape
    m_per = m_total // N_DEV

    def body(x_ref, w_hbm, out_ref,
             xs_ref, xg_ref, w_buf, send_sems, recv_sems, w_sems):
        my = lax.axis_index("i")

        barrier = pltpu.get_barrier_semaphore()
        for d in range(1, N_DEV):
            pl.semaphore_signal(
                barrier, inc=1,
                device_id=((my + d) % N_DEV,),
                device_id_type=pl.DeviceIdType.MESH,
            )
        pl.semaphore_wait(barrier, N_DEV - 1)

        xs_ref[...] = x_ref[...].astype(jnp.bfloat16)
        xg_ref[0] = xs_ref[pl.ds(my * m_per, m_per), :]

        rdmas = []
        for d in range(1, N_DEV):
            tgt = (my + d) % N_DEV
            rdma = pltpu.make_async_remote_copy(
                src_ref=xs_ref.at[pl.ds(tgt * m_per, m_per), :],
                dst_ref=xg_ref.at[N_DEV - d],
                send_sem=send_sems.at[d - 1],
                recv_sem=recv_sems.at[d - 1],
                device_id=(tgt,),
                device_id_type=pl.DeviceIdType.MESH,
            )
            rdma.start()
            rdmas.append(rdma)

        n_c = n // N_CHUNK

        def fetch_w(sidx, h, slot):
            kb = (my + sidx) % N_DEV
            cp = pltpu.make_async_copy(
                w_hbm.at[pl.ds(kb * k_shard, k_shard), pl.ds(h * n_c, n_c)],
                w_buf.at[slot],
                w_sems.at[slot],
            )
            cp.start()
            return cp

        steps = [(s, h) for s in range(N_DEV) for h in range(N_CHUNK)]
        w_cp = fetch_w(0, 0, 0)
        for idx, (s, h) in enumerate(steps):
            if s > 0 and h == 0:
                rdmas[(N_DEV - s) - 1].wait_recv()
            w_cp.wait()
            w_bf = w_buf[idx % 2].astype(jnp.bfloat16)
            if idx + 1 < len(steps):
                s2, h2 = steps[idx + 1]
                w_cp = fetch_w(s2, h2, (idx + 1) % 2)
            contrib = jnp.dot(
                xg_ref[s], w_bf,
                preferred_element_type=jnp.float32,
            )
            if s == 0:
                out_ref[:, pl.ds(h * n_c, n_c)] = contrib
            else:
                out_ref[:, pl.ds(h * n_c, n_c)] += contrib

        out_ref[...] = jnp.maximum(out_ref[...], 0.0)

        for rdma in rdmas:
            rdma.wait_send()

    return pl.pallas_call(
        body,
        out_shape=jax.ShapeDtypeStruct((m_per, n), jnp.float32),
        in_specs=[
            pl.BlockSpec(memory_space=pltpu.VMEM),
            pl.BlockSpec(memory_space=pltpu.MemorySpace.HBM),
        ],
        out_specs=pl.BlockSpec(memory_space=pltpu.VMEM),
        scratch_shapes=[
            pltpu.VMEM((m_total, k_shard), jnp.bfloat16),
            pltpu.VMEM((N_DEV, m_per, k_shard), jnp.bfloat16),
            pltpu.VMEM((2, k_shard, n // N_CHUNK), jnp.float32),
            pltpu.SemaphoreType.DMA((N_DEV - 1,)),
            pltpu.SemaphoreType.DMA((N_DEV - 1,)),
            pltpu.SemaphoreType.DMA((2,)),
        ],
        compiler_params=pltpu.CompilerParams(
            collective_id=0,
            vmem_limit_bytes=100 * 1024 * 1024,
        ),
    )(x, w_mat)


# device time: 92968 ns/iter; 1.5061x vs baseline; 1.5061x over previous
import os

import jax
import jax.numpy as jnp
from jax import lax
from jax.experimental import pallas as pl
from jax.experimental.pallas import tpu as pltpu

N_DEV = 8
N_C = 512
N_BUF = 3
_MODE = os.environ.get("KMODE", "full")


def kernel(x, w_mat):
    m_total, k_shard = x.shape
    k_total, n = w_mat.shape
    m_per = m_total // N_DEV
    n_chunks = n // N_C

    def body(x_ref, w_hbm, out_ref,
             xs_ref, xg_ref, w_buf, send_sems, recv_sems, w_sems):
        my = lax.axis_index("i")

        barrier = pltpu.get_barrier_semaphore()
        for d in range(1, N_DEV):
            pl.semaphore_signal(
                barrier, inc=1,
                device_id=((my + d) % N_DEV,),
                device_id_type=pl.DeviceIdType.MESH,
            )
        pl.semaphore_wait(barrier, N_DEV - 1)

        xs_ref[...] = x_ref[...].astype(jnp.bfloat16)
        my_col = my * k_shard
        xg_ref[:, pl.ds(my_col, k_shard)] = xs_ref[pl.ds(my * m_per, m_per), :]

        rdmas = []
        if _MODE != "nocomm":
            for d in range(1, N_DEV):
                tgt = (my + d) % N_DEV
                rdma = pltpu.make_async_remote_copy(
                    src_ref=xs_ref.at[pl.ds(tgt * m_per, m_per), :],
                    dst_ref=xg_ref.at[:, pl.ds(my_col, k_shard)],
                    send_sem=send_sems.at[d - 1],
                    recv_sem=recv_sems.at[d - 1],
                    device_id=(tgt,),
                    device_id_type=pl.DeviceIdType.MESH,
                )
                rdma.start()
                rdmas.append(rdma)

        def fetch_w(c):
            cp = pltpu.make_async_copy(
                w_hbm.at[:, pl.ds(c * N_C, N_C)],
                w_buf.at[c % N_BUF],
                w_sems.at[c % N_BUF],
            )
            cp.start()
            return cp

        cps = {}
        if _MODE != "nodma":
            for c in range(min(N_BUF, n_chunks)):
                cps[c] = fetch_w(c)

        for r in rdmas:
            r.wait_recv()

        for c in range(n_chunks):
            if _MODE != "nodma":
                cps.pop(c).wait()
            if _MODE == "nodot":
                if c + N_BUF < n_chunks:
                    cps[c + N_BUF] = fetch_w(c + N_BUF)
                continue
            w_bf = w_buf[c % N_BUF].astype(jnp.bfloat16)
            if _MODE != "nodma" and c + N_BUF < n_chunks:
                cps[c + N_BUF] = fetch_w(c + N_BUF)
            acc = jnp.dot(
                xg_ref[...], w_bf,
                preferred_element_type=jnp.float32,
            )
            out_ref[:, pl.ds(c * N_C, N_C)] = jnp.maximum(acc, 0.0).astype(
                jnp.bfloat16
            )

        if _MODE == "nodot":
            out_ref[...] = jnp.zeros_like(out_ref)

        for rdma in rdmas:
            rdma.wait_send()

    return pl.pallas_call(
        body,
        out_shape=jax.ShapeDtypeStruct((m_per, n), jnp.bfloat16),
        in_specs=[
            pl.BlockSpec(memory_space=pltpu.VMEM),
            pl.BlockSpec(memory_space=pltpu.MemorySpace.HBM),
        ],
        out_specs=pl.BlockSpec(memory_space=pltpu.VMEM),
        scratch_shapes=[
            pltpu.VMEM((m_total, k_shard), jnp.bfloat16),
            pltpu.VMEM((m_per, k_total), jnp.bfloat16),
            pltpu.VMEM((N_BUF, k_total, N_C), jnp.float32),
            pltpu.SemaphoreType.DMA((N_DEV - 1,)),
            pltpu.SemaphoreType.DMA((N_DEV - 1,)),
            pltpu.SemaphoreType.DMA((N_BUF,)),
        ],
        compiler_params=pltpu.CompilerParams(
            collective_id=0,
            vmem_limit_bytes=100 * 1024 * 1024,
        ),
    )(x, w_mat)


# device time: 77024 ns/iter; 1.8178x vs baseline; 1.2070x over previous
import jax
import jax.numpy as jnp
from jax import lax
from jax.experimental import pallas as pl
from jax.experimental.pallas import tpu as pltpu

N_DEV = 8
N_C = 1024
N_BUF = 3
PHASES = ((0, 1), (1, 3), (4, 4))


def kernel(x, w_mat):
    m_total, k_shard = x.shape
    k_total, n = w_mat.shape
    m_per = m_total // N_DEV
    n_chunks = n // N_C

    def body(x_ref, w_hbm, out_ref,
             xs_ref, xg_ref, w_buf, send_sems, recv_sems, w_sems):
        my = lax.axis_index("i")

        barrier = pltpu.get_barrier_semaphore()
        for d in range(1, N_DEV):
            pl.semaphore_signal(
                barrier, inc=1,
                device_id=((my + d) % N_DEV,),
                device_id_type=pl.DeviceIdType.MESH,
            )
        pl.semaphore_wait(barrier, N_DEV - 1)

        xs_ref[...] = x_ref[...].astype(jnp.bfloat16)
        xg_ref[:, pl.ds(0, k_shard)] = xs_ref[pl.ds(my * m_per, m_per), :]

        def make_rdma(d):
            tgt = (my + d) % N_DEV
            return pltpu.make_async_remote_copy(
                src_ref=xs_ref.at[pl.ds(tgt * m_per, m_per), :],
                dst_ref=xg_ref.at[:, pl.ds(d * k_shard, k_shard)],
                send_sem=send_sems.at[d - 1],
                recv_sem=recv_sems.at[d - 1],
                device_id=(tgt,),
                device_id_type=pl.DeviceIdType.MESH,
            )

        rdmas = [make_rdma(d) for d in range(1, N_DEV)]
        for d in (1, 2, 3):
            rdmas[d - 1].start()

        jobs = [(p, c) for p in range(len(PHASES)) for c in range(n_chunks)]

        def fetch(job_idx):
            p, c = jobs[job_idx]
            jb0, nb = PHASES[p]
            slot = job_idx % N_BUF
            cps = []
            for i in range(nb):
                src_blk = (my + (N_DEV - (jb0 + i))) % N_DEV
                cp = pltpu.make_async_copy(
                    w_hbm.at[pl.ds(src_blk * k_shard, k_shard),
                             pl.ds(c * N_C, N_C)],
                    w_buf.at[slot, pl.ds(i * k_shard, k_shard), :],
                    w_sems.at[slot],
                )
                cp.start()
                cps.append(cp)
            return cps

        inflight = {}
        for j in range(N_BUF):
            inflight[j] = fetch(j)

        hi_started = False
        for job_idx, (p, c) in enumerate(jobs):
            jb0, nb = PHASES[p]
            kk = nb * k_shard
            if p == 1 and c == 0:
                for d in (1, 2, 3):
                    rdmas[d - 1].wait_recv()
            if p == 2 and c == 0:
                for d in (4, 5, 6, 7):
                    rdmas[d - 1].wait_recv()
            for cp in inflight.pop(job_idx):
                cp.wait()
            rhs = w_buf[job_idx % N_BUF, pl.ds(0, kk), :].astype(jnp.bfloat16)
            if job_idx + N_BUF < len(jobs):
                inflight[job_idx + N_BUF] = fetch(job_idx + N_BUF)
            acc = jnp.dot(
                xg_ref[:, pl.ds(jb0 * k_shard, kk)], rhs,
                preferred_element_type=jnp.float32,
            )
            cs = pl.ds(c * N_C, N_C)
            if p == 0:
                out_ref[:, cs] = acc.astype(jnp.bfloat16)
            elif p == 1:
                out_ref[:, cs] = (
                    out_ref[:, cs].astype(jnp.float32) + acc
                ).astype(jnp.bfloat16)
            else:
                out_ref[:, cs] = jnp.maximum(
                    out_ref[:, cs].astype(jnp.float32) + acc, 0.0
                ).astype(jnp.bfloat16)
            if p == 0 and c == 3 and not hi_started:
                for d in (1, 2, 3):
                    rdmas[d - 1].wait_send()
                for d in (4, 5, 6, 7):
                    rdmas[d - 1].start()
                hi_started = True

        for d in (4, 5, 6, 7):
            rdmas[d - 1].wait_send()

    return pl.pallas_call(
        body,
        out_shape=jax.ShapeDtypeStruct((m_per, n), jnp.bfloat16),
        in_specs=[
            pl.BlockSpec(memory_space=pltpu.VMEM),
            pl.BlockSpec(memory_space=pltpu.MemorySpace.HBM),
        ],
        out_specs=pl.BlockSpec(memory_space=pltpu.VMEM),
        scratch_shapes=[
            pltpu.VMEM((m_total, k_shard), jnp.bfloat16),
            pltpu.VMEM((m_per, k_total), jnp.bfloat16),
            pltpu.VMEM((N_BUF, 4 * k_shard, N_C), jnp.float32),
            pltpu.SemaphoreType.DMA((N_DEV - 1,)),
            pltpu.SemaphoreType.DMA((N_DEV - 1,)),
            pltpu.SemaphoreType.DMA((N_BUF,)),
        ],
        compiler_params=pltpu.CompilerParams(
            collective_id=0,
            vmem_limit_bytes=100 * 1024 * 1024,
        ),
    )(x, w_mat)
